# baseline (device time: 1367039 ns/iter reference)
import jax
import jax.numpy as jnp
from jax import lax
from jax.experimental import pallas as pl
from jax.experimental.pallas import tpu as pltpu

NY = 4
NZ = 4
CH = 256


def kernel(partial, resid, gamma):
    _, M, D = partial.shape
    HALF = M // 2
    NC = HALF // CH
    gamma2d = gamma.reshape(1, D)
    MESH = pl.DeviceIdType.MESH

    def body(partial_ref, resid_ref, gamma_ref, out_ref,
             pre_in, pre_out, suf_in, suf_out,
             va, vb, vsum, vout,
             csem, pre_s, pre_r, suf_s, suf_r, z0_s, z0_r, z3_s, z3_r):
        my_x = lax.axis_index("x")
        my_y = lax.axis_index("y")
        my_z = lax.axis_index("z")
        is_z0 = my_z == 0
        is_z3 = my_z == NZ - 1
        is_edge = jnp.logical_or(is_z0, is_z3)
        is_zmid = jnp.logical_not(is_edge)
        is_ymid = jnp.logical_and(my_y > 0, my_y < NY - 1)
        base = jnp.where(is_z0, 0, HALF)
        yr = (my_x, my_y + 1, my_z)
        yl = (my_x, my_y - 1, my_z)
        zdn = (my_x, my_y, my_z + 1)
        zup = (my_x, my_y, my_z - 1)
        me = (my_x, my_y, my_z)

        def rwait(sem, dst):
            pltpu.make_async_remote_copy(
                src_ref=dst, dst_ref=dst, send_sem=sem, recv_sem=sem,
                device_id=me, device_id_type=MESH).wait_recv()

        def swait(sem):
            pltpu.make_async_remote_copy(
                src_ref=vout, dst_ref=vout, send_sem=sem, recv_sem=sem,
                device_id=me, device_id_type=MESH).wait_send()

        def copy(src, dst, i):
            cp = pltpu.make_async_copy(src, dst, csem.at[i])
            cp.start()
            return cp

        for c in range(NC):
            rc = pl.ds(c * CH, CH)
            grows = pl.ds(base + c * CH, CH)

            @pl.when(jnp.logical_and(is_edge, my_y == 0))
            def _():
                pltpu.make_async_remote_copy(
                    src_ref=partial_ref.at[0, grows, :],
                    dst_ref=pre_in.at[rc, :],
                    send_sem=pre_s.at[c], recv_sem=pre_r.at[c],
                    device_id=yr, device_id_type=MESH).start()

            @pl.when(jnp.logical_and(is_edge, my_y == NY - 1))
            def _():
                pltpu.make_async_remote_copy(
                    src_ref=partial_ref.at[0, grows, :],
                    dst_ref=suf_in.at[rc, :],
                    send_sem=suf_s.at[c], recv_sem=suf_r.at[c],
                    device_id=yl, device_id_type=MESH).start()

        for c in range(NC):
            rc = pl.ds(c * CH, CH)
            grows = pl.ds(base + c * CH, CH)

            @pl.when(jnp.logical_and(is_edge, my_y > 0))
            def _():
                rwait(pre_r.at[c], pre_in.at[rc, :])

            @pl.when(jnp.logical_and(is_edge, is_ymid))
            def _():
                ca = copy(pre_in.at[rc, :], va, 0)
                cb = copy(partial_ref.at[0, grows, :], vb, 1)
                ca.wait()
                cb.wait()
                vsum[...] = va[...] + vb[...]
                copy(vsum, pre_out.at[rc, :], 2).wait()
                pltpu.make_async_remote_copy(
                    src_ref=pre_out.at[rc, :], dst_ref=pre_in.at[rc, :],
                    send_sem=pre_s.at[c], recv_sem=pre_r.at[c],
                    device_id=yr, device_id_type=MESH).start()

            @pl.when(jnp.logical_and(is_edge, my_y < NY - 1))
            def _():
                rwait(suf_r.at[c], suf_in.at[rc, :])

            @pl.when(jnp.logical_and(is_edge, is_ymid))
            def _():
                ca = copy(suf_in.at[rc, :], va, 0)
                cb = copy(partial_ref.at[0, grows, :], vb, 1)
                ca.wait()
                cb.wait()
                vsum[...] = va[...] + vb[...]
                copy(vsum, suf_out.at[rc, :], 2).wait()
                pltpu.make_async_remote_copy(
                    src_ref=suf_out.at[rc, :], dst_ref=suf_in.at[rc, :],
                    send_sem=suf_s.at[c], recv_sem=suf_r.at[c],
                    device_id=yl, device_id_type=MESH).start()

            @pl.when(is_edge)
            def _():
                ca = copy(partial_ref.at[0, grows, :], va, 0)
                cb = copy(resid_ref.at[grows, :], vb, 1)
                ca.wait()
                cb.wait()
                vsum[...] = va[...] + vb[...]

            @pl.when(jnp.logical_and(is_edge, my_y > 0))
            def _():
                copy(pre_in.at[rc, :], va, 0).wait()
                vsum[...] = vsum[...] + va[...]

            @pl.when(jnp.logical_and(is_edge, my_y < NY - 1))
            def _():
                copy(suf_in.at[rc, :], vb, 1).wait()
                vsum[...] = vsum[...] + vb[...]

            @pl.when(is_edge)
            def _():
                y = vsum[...]
                rms = jnp.sqrt(jnp.mean(y * y, axis=-1, keepdims=True) + 1e-6)
                vout[...] = y / rms * gamma_ref[...]
                copy(vout, out_ref.at[grows, :], 2).wait()

            @pl.when(is_z0)
            def _():
                pltpu.make_async_remote_copy(
                    src_ref=out_ref.at[grows, :], dst_ref=out_ref.at[grows, :],
                    send_sem=z0_s.at[c], recv_sem=z0_r.at[c],
                    device_id=zdn, device_id_type=MESH).start()

            @pl.when(is_z3)
            def _():
                pltpu.make_async_remote_copy(
                    src_ref=out_ref.at[grows, :], dst_ref=out_ref.at[grows, :],
                    send_sem=z3_s.at[c], recv_sem=z3_r.at[c],
                    device_id=zup, device_id_type=MESH).start()

        for c in range(NC):
            r0 = pl.ds(c * CH, CH)
            r3 = pl.ds(HALF + c * CH, CH)

            @pl.when(is_zmid)
            def _():
                rwait(z0_r.at[c], out_ref.at[r0, :])
                pltpu.make_async_remote_copy(
                    src_ref=out_ref.at[r0, :], dst_ref=out_ref.at[r0, :],
                    send_sem=z0_s.at[c], recv_sem=z0_r.at[c],
                    device_id=zdn, device_id_type=MESH).start()
                rwait(z3_r.at[c], out_ref.at[r3, :])
                pltpu.make_async_remote_copy(
                    src_ref=out_ref.at[r3, :], dst_ref=out_ref.at[r3, :],
                    send_sem=z3_s.at[c], recv_sem=z3_r.at[c],
                    device_id=zup, device_id_type=MESH).start()

        for c in range(NC):
            r0 = pl.ds(c * CH, CH)
            r3 = pl.ds(HALF + c * CH, CH)

            @pl.when(is_z0)
            def _():
                rwait(z3_r.at[c], out_ref.at[r3, :])

            @pl.when(is_z3)
            def _():
                rwait(z0_r.at[c], out_ref.at[r0, :])

        for c in range(NC):
            @pl.when(jnp.logical_and(is_edge, my_y < NY - 1))
            def _():
                swait(pre_s.at[c])

            @pl.when(jnp.logical_and(is_edge, my_y > 0))
            def _():
                swait(suf_s.at[c])

            @pl.when(jnp.logical_or(is_z0, is_zmid))
            def _():
                swait(z0_s.at[c])

            @pl.when(jnp.logical_or(is_z3, is_zmid))
            def _():
                swait(z3_s.at[c])

    out, _, _, _, _ = pl.pallas_call(
        body,
        out_shape=[
            jax.ShapeDtypeStruct((M, D), jnp.float32),
            jax.ShapeDtypeStruct((HALF, D), jnp.float32),
            jax.ShapeDtypeStruct((HALF, D), jnp.float32),
            jax.ShapeDtypeStruct((HALF, D), jnp.float32),
            jax.ShapeDtypeStruct((HALF, D), jnp.float32),
        ],
        in_specs=[
            pl.BlockSpec(memory_space=pl.ANY),
            pl.BlockSpec(memory_space=pl.ANY),
            pl.BlockSpec(memory_space=pltpu.MemorySpace.VMEM),
        ],
        out_specs=[pl.BlockSpec(memory_space=pl.ANY)] * 5,
        scratch_shapes=[
            pltpu.VMEM((CH, D), jnp.float32),
            pltpu.VMEM((CH, D), jnp.float32),
            pltpu.VMEM((CH, D), jnp.float32),
            pltpu.VMEM((CH, D), jnp.float32),
            pltpu.SemaphoreType.DMA((3,)),
            pltpu.SemaphoreType.DMA((NC,)),
            pltpu.SemaphoreType.DMA((NC,)),
            pltpu.SemaphoreType.DMA((NC,)),
            pltpu.SemaphoreType.DMA((NC,)),
            pltpu.SemaphoreType.DMA((NC,)),
            pltpu.SemaphoreType.DMA((NC,)),
            pltpu.SemaphoreType.DMA((NC,)),
            pltpu.SemaphoreType.DMA((NC,)),
        ],
        compiler_params=pltpu.CompilerParams(
            vmem_limit_bytes=60 * 1024 * 1024),
    )(partial, resid, gamma2d)
    return out
